# baseline (speedup 1.0000x reference)
import jax
import jax.numpy as jnp
from jax import lax
from jax.experimental import pallas as pl
from jax.experimental.pallas import tpu as pltpu
from jax.experimental.pallas import tpu_sc as plsc

NUM_EMB = 400
EMB_DIM = 32
C_IN = 16
C_OUT = 78
BATCH = 16384
HIST = 200

NC = 2
NS = 16
NW = NC * NS

BB = BATCH // NW
BC = 256
NREP = 4
WSTR = 33


N_CHUNK = 2 * HIST


def _sc_body(x_hbm, w_hbm, out_hbm, wv, xvs, ovs, isems, osems):
    wid = lax.axis_index("s") * NC + lax.axis_index("c")
    b0 = wid * BB

    pltpu.sync_copy(w_hbm, wv)

    lane = lax.iota(jnp.int32, 16)
    loff = (lane % NREP) * WSTR

    def in_dma(ci, p):
        h, half = ci // 2, ci % 2
        return pltpu.make_async_copy(
            x_hbm.at[pl.ds(h * C_IN, C_IN), pl.ds(b0 + half * BC, BC)],
            xvs[p], isems[p])

    def out_dma(ci, p):
        h, half = ci // 2, ci % 2
        return pltpu.make_async_copy(
            ovs[p],
            out_hbm.at[pl.ds(0, C_OUT), pl.ds(h, 1),
                       pl.ds(b0 + half * BC, BC)],
            osems[p])

    def compute(xv, ov):
        @plsc.parallel_loop(0, BC, step=16, unroll=4)
        def _group(k):
            sl = pl.ds(k, 16)
            ia = jnp.clip(xv[0, sl].astype(jnp.int32), 0, NUM_EMB - 1)
            io = jnp.clip(xv[8, sl].astype(jnp.int32), 0, NUM_EMB - 1)
            ba = ia * (NREP * WSTR) + loff
            bo = io * (NREP * WSTR) + loff
            for j in range(EMB_DIM):
                ov[j, 0, sl] = plsc.load_gather(wv, [ba + j])
                ov[EMB_DIM + 7 + j, 0, sl] = plsc.load_gather(wv, [bo + j])
            for c in range(1, 8):
                ov[EMB_DIM + c - 1, 0, sl] = xv[c, sl]
                ov[2 * EMB_DIM + 6 + c, 0, sl] = xv[c + 8, sl]

    in_dma(0, 0).start()
    in_dma(1, 1).start()

    @pl.loop(0, N_CHUNK // 2)
    def _cpair(i):
        for p in range(2):
            ci = 2 * i + p
            in_dma(ci, p).wait()

            @pl.when(ci >= 2)
            def _():
                out_dma(ci - 2, p).wait()

            compute(xvs[p], ovs[p])
            out_dma(ci, p).start()

            @pl.when(ci + 2 < N_CHUNK)
            def _():
                in_dma(ci + 2, p).start()

    out_dma(N_CHUNK - 2, 0).wait()
    out_dma(N_CHUNK - 1, 1).wait()


@jax.jit
def _sc_embed(x2, w):
    run = pl.kernel(
        _sc_body,
        out_type=jax.ShapeDtypeStruct((C_OUT, HIST, BATCH), jnp.float32),
        mesh=plsc.VectorSubcoreMesh(core_axis_name="c", subcore_axis_name="s"),
        scratch_types=[
            pltpu.VMEM((NUM_EMB * NREP * WSTR,), jnp.float32),
            [pltpu.VMEM((C_IN, BC), jnp.float32)] * 2,
            [pltpu.VMEM((C_OUT, 1, BC), jnp.float32)] * 2,
            [pltpu.SemaphoreType.DMA] * 2,
            [pltpu.SemaphoreType.DMA] * 2,
        ],
        compiler_params=pltpu.CompilerParams(
            use_tc_tiling_on_sc=False, needs_layout_passes=False),
    )
    return run(x2, w)


def kernel(x, W):
    x_t = jnp.transpose(x, (1, 2, 0)).reshape(HIST * C_IN, BATCH)
    pad = jnp.zeros((NUM_EMB, WSTR - EMB_DIM), W.dtype)
    w_rep = jnp.concatenate([W, pad] * NREP, axis=1).reshape(-1)
    out_t = _sc_embed(x_t, w_rep)
    return jnp.transpose(out_t, (2, 1, 0))
